# baseline (speedup 1.0000x reference)
import functools

import jax
import jax.numpy as jnp
from jax import lax
from jax.experimental import pallas as pl
from jax.experimental.pallas import tpu as pltpu
from jax.experimental.pallas import tpu_sc as plsc

_N = 10000
_E = 320000
_D = 128
_NG = 64
_NC = 2
_NS = 16
_CH = 80
_ET = _E // _NS
_NCHUNK = _ET // _CH
_RCH = 80
_NROWCH = _N // _RCH
_ROW_ITERS = -(-_NROWCH // _NS)

_CHB = 128
_NBUF = 4
_CPT = 160
_EPT = _CPT * _CHB
_EP = _NS * _EPT
_NPAD = 16
_NROUND = _CPT // _NBUF

_HIGHEST = lax.Precision.HIGHEST


def _dot_bf16(a, b):
    return jnp.dot(a.astype(jnp.bfloat16), b.astype(jnp.bfloat16),
                   preferred_element_type=jnp.float32)

_sc_mesh = plsc.VectorSubcoreMesh(core_axis_name="c", subcore_axis_name="s")


def _sc_counts(dst, bat, ones_rows, zeros_rows):
    @functools.partial(
        pl.kernel,
        out_type=[
            jax.ShapeDtypeStruct((_NC * _N, _D), jnp.float32),
            jax.ShapeDtypeStruct((_NC * _NG, _D), jnp.float32),
        ],
        mesh=_sc_mesh,
        scratch_types=[
            pltpu.VMEM((_CHB,), jnp.int32),
            pltpu.VMEM((_CHB,), jnp.int32),
            pltpu.VMEM((_CHB,), jnp.int32),
            pltpu.VMEM((_CHB,), jnp.int32),
            pltpu.VMEM((_CH,), jnp.int32),
            pltpu.VMEM((_CHB, _D), jnp.float32),
            pltpu.VMEM_SHARED((_N + _NPAD, _D), jnp.float32),
            pltpu.VMEM_SHARED((_NG, _D), jnp.float32),
            pltpu.SemaphoreType.DMA,
            pltpu.SemaphoreType.DMA,
            pltpu.SemaphoreType.DMA,
            pltpu.SemaphoreType.DMA,
            pltpu.SemaphoreType.DMA,
            pltpu.SemaphoreType.DMA,
            pltpu.SemaphoreType.DMA,
            pltpu.SemaphoreType.DMA,
        ],
    )
    def k(dst_hbm, bat_hbm, ones_hbm, zer_hbm, cnt_hbm, bcnt_hbm,
          di0, di1, di2, di3, bidx, ones_v, cnt_s, bcnt_s,
          smi0, smi1, smi2, smi3, ss0, ss1, ss2, ss3):
        didx = (di0, di1, di2, di3)
        semi = (smi0, smi1, smi2, smi3)
        sems = (ss0, ss1, ss2, ss3)
        c = lax.axis_index("c")
        t = lax.axis_index("s")
        pltpu.sync_copy(zer_hbm, ones_v)

        @pl.loop(0, _ROW_ITERS)
        def _(i):
            ch = i * _NS + t

            @pl.when(ch < _NROWCH)
            def _():
                pltpu.sync_copy(ones_v.at[pl.ds(0, _RCH)],
                                cnt_s.at[pl.ds(ch * _RCH, _RCH)])

        @pl.when(t == 0)
        def _():
            pltpu.sync_copy(ones_v.at[pl.ds(0, _NG)], bcnt_s)
            pltpu.sync_copy(ones_v.at[pl.ds(0, _NPAD)],
                            cnt_s.at[pl.ds(_N, _NPAD)])

        pltpu.sync_copy(ones_hbm, ones_v)
        plsc.subcore_barrier()

        base_e = c * _EP + t * _EPT

        def start_idx(ch_i, j):
            pltpu.async_copy(dst_hbm.at[pl.ds(base_e + ch_i * _CHB, _CHB)],
                             didx[j], semi[j])

        def wait_idx(j):
            pltpu.make_async_copy(dst_hbm.at[pl.ds(0, _CHB)], didx[j],
                                  semi[j]).wait()

        def start_scatter(b):
            pltpu.async_copy(ones_v, cnt_s.at[didx[b]], sems[b], add=True)

        def wait_scatter(b):
            pltpu.make_async_copy(ones_v, cnt_s.at[didx[b]], sems[b]).wait()

        for b in range(4):
            start_idx(b, b)

        _G = _CPT // 4

        @pl.loop(0, _G)
        def _(g):
            for b in range(4):
                wait_idx(b)
                start_scatter(b)
            for b in range(4):
                wait_scatter(b)

                @pl.when(g < _G - 1)
                def _(b=b):
                    start_idx((g + 1) * 4 + b, b)

        @pl.loop(0, _ROW_ITERS)
        def _(i):
            ch = i * _NS + t

            @pl.when(ch < _NROWCH)
            def _():
                pltpu.sync_copy(bat_hbm.at[pl.ds(c * _N + ch * _CH, _CH)], bidx)
                pltpu.sync_copy(ones_v.at[pl.ds(0, _CH)], bcnt_s.at[bidx],
                                add=True)

        plsc.subcore_barrier()

        @pl.loop(0, _ROW_ITERS)
        def _(i):
            ch = i * _NS + t

            @pl.when(ch < _NROWCH)
            def _():
                pltpu.sync_copy(cnt_s.at[pl.ds(ch * _RCH, _RCH)],
                                cnt_hbm.at[pl.ds(c * _N + ch * _RCH, _RCH)])

        @pl.when(t == 0)
        def _():
            pltpu.sync_copy(bcnt_s, bcnt_hbm.at[pl.ds(c * _NG, _NG)])

    return k(dst, bat, ones_rows, zeros_rows)


def _sc_conv(xs_flat, src_pad, dst_pad, zeros_rows):
    @functools.partial(
        pl.kernel,
        out_type=jax.ShapeDtypeStruct((_NC * _N, _D), jnp.float32),
        mesh=_sc_mesh,
        scratch_types=[
            pltpu.VMEM((_CHB,), jnp.int32),
            pltpu.VMEM((_CHB,), jnp.int32),
            pltpu.VMEM((_CHB,), jnp.int32),
            pltpu.VMEM((_CHB,), jnp.int32),
            pltpu.VMEM((_CHB,), jnp.int32),
            pltpu.VMEM((_CHB,), jnp.int32),
            pltpu.VMEM((_CHB,), jnp.int32),
            pltpu.VMEM((_CHB,), jnp.int32),
            pltpu.VMEM((_CHB, _D), jnp.float32),
            pltpu.VMEM((_CHB, _D), jnp.float32),
            pltpu.VMEM_SHARED((_N + _NPAD, _D), jnp.float32),
            pltpu.SemaphoreType.DMA,
            pltpu.SemaphoreType.DMA,
            pltpu.SemaphoreType.DMA,
            pltpu.SemaphoreType.DMA,
            pltpu.SemaphoreType.DMA,
            pltpu.SemaphoreType.DMA,
            pltpu.SemaphoreType.DMA,
            pltpu.SemaphoreType.DMA,
        ],
    )
    def k(xs_hbm, src_hbm, dst_hbm, zer_hbm, acc_hbm,
          si0, si1, si2, si3, di0, di1, di2, di3, r0, r1, acc_s,
          smi0, smi1, smi2, smi3, sg0, sg1, ss0, ss1):
        sidx = (si0, si1, si2, si3)
        didx = (di0, di1, di2, di3)
        rows = (r0, r1)
        semi = (smi0, smi1, smi2, smi3)
        semg = (sg0, sg1)
        sems = (ss0, ss1)
        c = lax.axis_index("c")
        t = lax.axis_index("s")
        pltpu.sync_copy(zer_hbm, r0)

        @pl.loop(0, _ROW_ITERS)
        def _(i):
            ch = i * _NS + t

            @pl.when(ch < _NROWCH)
            def _():
                pltpu.sync_copy(r0.at[pl.ds(0, _RCH)],
                                acc_s.at[pl.ds(ch * _RCH, _RCH)])

        @pl.when(t == 0)
        def _():
            pltpu.sync_copy(r0.at[pl.ds(0, _NPAD)],
                            acc_s.at[pl.ds(_N, _NPAD)])

        plsc.subcore_barrier()

        base_e = c * _EP + t * _EPT

        def start_idx(ch_i, j):
            off = base_e + ch_i * _CHB
            pltpu.async_copy(src_hbm.at[pl.ds(off, _CHB)], sidx[j], semi[j])
            pltpu.async_copy(dst_hbm.at[pl.ds(off, _CHB)], didx[j], semi[j])

        def wait_idx(j):
            pltpu.make_async_copy(src_hbm.at[pl.ds(0, _CHB)], sidx[j],
                                  semi[j]).wait()
            pltpu.make_async_copy(dst_hbm.at[pl.ds(0, _CHB)], didx[j],
                                  semi[j]).wait()

        def start_gather(j, b):
            pltpu.async_copy(xs_hbm.at[sidx[j]], rows[b], semg[b])

        def wait_gather(j, b):
            pltpu.make_async_copy(xs_hbm.at[sidx[j]], rows[b], semg[b]).wait()

        def start_scatter(j, b):
            pltpu.async_copy(rows[b], acc_s.at[didx[j]], sems[b], add=True)

        def wait_scatter(j, b):
            pltpu.make_async_copy(rows[b], acc_s.at[didx[j]], sems[b]).wait()

        for j in range(4):
            start_idx(j, j)
        wait_idx(0)
        start_gather(0, 0)
        wait_idx(1)
        start_gather(1, 1)

        _G = _CPT // 4

        @pl.loop(0, _G)
        def _(g):
            q = g * 4
            wait_gather(0, 0)
            start_scatter(0, 0)
            wait_gather(1, 1)
            start_scatter(1, 1)
            wait_scatter(0, 0)

            @pl.when(g < _G - 1)
            def _():
                start_idx(q + 4, 0)

            wait_idx(2)
            start_gather(2, 0)
            wait_scatter(1, 1)

            @pl.when(g < _G - 1)
            def _():
                start_idx(q + 5, 1)

            wait_idx(3)
            start_gather(3, 1)
            wait_gather(2, 0)
            start_scatter(2, 0)
            wait_gather(3, 1)
            start_scatter(3, 1)
            wait_scatter(2, 0)

            @pl.when(g < _G - 1)
            def _():
                start_idx(q + 6, 2)
                wait_idx(0)
                start_gather(0, 0)

            wait_scatter(3, 1)

            @pl.when(g < _G - 1)
            def _():
                start_idx(q + 7, 3)
                wait_idx(1)
                start_gather(1, 1)

        plsc.subcore_barrier()

        @pl.loop(0, _ROW_ITERS)
        def _(i):
            ch = i * _NS + t

            @pl.when(ch < _NROWCH)
            def _():
                pltpu.sync_copy(acc_s.at[pl.ds(ch * _RCH, _RCH)],
                                acc_hbm.at[pl.ds(c * _N + ch * _RCH, _RCH)])

    return k(xs_flat, src_pad, dst_pad, zeros_rows)


_BLK = 2000
_NBLK = _NC * _N // _BLK
_TBLK = _N // _BLK

_row_spec = lambda w: pl.BlockSpec((_BLK, w), lambda i: (i, 0))
_pair_spec2 = pl.BlockSpec((1, 1, _D), lambda i: (i // _TBLK, 0, 0))
_pair_spec3 = pl.BlockSpec((1, _D, _D), lambda i: (i // _TBLK, 0, 0))


def _tc_matmul(x_flat, w_pair):
    def body(x_ref, w_ref, o_ref):
        o_ref[...] = _dot_bf16(x_ref[...], w_ref[0].T)

    return pl.pallas_call(
        body,
        grid=(_NBLK,),
        in_specs=[_row_spec(_D), _pair_spec3],
        out_specs=_row_spec(_D),
        out_shape=jax.ShapeDtypeStruct((_NC * _N, _D), jnp.float32),
    )(x_flat, w_pair)


def _tc_scale(cnt, h_flat):
    def body(cnt_ref, h_ref, xs_ref, dis_ref):
        dis = lax.rsqrt(cnt_ref[:, 0:1] + 1.0)
        dis_ref[...] = dis
        xs_ref[...] = h_ref[...] * dis

    return pl.pallas_call(
        body,
        grid=(_NBLK,),
        in_specs=[_row_spec(_D), _row_spec(_D)],
        out_specs=[_row_spec(_D), _row_spec(1)],
        out_shape=[
            jax.ShapeDtypeStruct((_NC * _N, _D), jnp.float32),
            jax.ShapeDtypeStruct((_NC * _N, 1), jnp.float32),
        ],
    )(cnt, h_flat)


def _tc_layer(acc_flat, xs_flat, dis, b_pair, w_pair):
    def body(acc_ref, xs_ref, dis_ref, b_ref, w_ref, o_ref):
        d = dis_ref[...]
        o = jax.nn.relu(d * (acc_ref[...] + xs_ref[...]) + b_ref[0])
        h2 = _dot_bf16(o, w_ref[0].T)
        o_ref[...] = d * h2

    return pl.pallas_call(
        body,
        grid=(_NBLK,),
        in_specs=[_row_spec(_D), _row_spec(_D), _row_spec(1),
                  _pair_spec2, _pair_spec3],
        out_specs=_row_spec(_D),
        out_shape=jax.ShapeDtypeStruct((_NC * _N, _D), jnp.float32),
    )(acc_flat, xs_flat, dis, b_pair, w_pair)


def _tc_pool(acc_flat, xs_flat, dis, b_pair, batf):
    def body(acc_ref, xs_ref, dis_ref, b_ref, bat_ref, s_ref):
        i = pl.program_id(0)
        d = dis_ref[...]
        o = jax.nn.relu(d * (acc_ref[...] + xs_ref[...]) + b_ref[0])
        seg = lax.broadcasted_iota(jnp.int32, (_NG, 1), 0).astype(jnp.float32)
        mask = (bat_ref[0] == seg).astype(jnp.float32)
        s = jnp.dot(mask, o, preferred_element_type=jnp.float32,
                    precision=_HIGHEST)

        @pl.when(i % _TBLK == 0)
        def _():
            s_ref[...] = jnp.zeros_like(s_ref)

        s_ref[0] += s

    return pl.pallas_call(
        body,
        grid=(_NBLK,),
        in_specs=[_row_spec(_D), _row_spec(_D), _row_spec(1), _pair_spec2,
                  pl.BlockSpec((1, 1, _BLK), lambda i: (i, 0, 0))],
        out_specs=pl.BlockSpec((1, _NG, _D), lambda i: (i // _TBLK, 0, 0)),
        out_shape=jax.ShapeDtypeStruct((_NC, _NG, _D), jnp.float32),
    )(acc_flat, xs_flat, dis, b_pair, batf)


def _tc_head(spool, bcnt, mW0, mb0, mW1, mb1, mW2, mb2, mW3, mb3):
    def body(s_ref, bcnt_ref, w0_ref, b0_ref, w1_ref, b1_ref, w2_ref, b2_ref,
             w3_ref, b3_ref, o_ref):
        g1 = s_ref[0] / jnp.maximum(bcnt_ref[0:_NG, 0:1], 1.0)
        g2 = s_ref[1] / jnp.maximum(bcnt_ref[_NG:2 * _NG, 0:1], 1.0)
        z = jnp.concatenate([g1, g2], axis=1)
        for w_ref, bias_ref, act in ((w0_ref, b0_ref, True), (w1_ref, b1_ref, True),
                                     (w2_ref, b2_ref, True), (w3_ref, b3_ref, False)):
            z = _dot_bf16(z, w_ref[...].T) + bias_ref[...][None, :]
            if act:
                z = jax.nn.relu(z)
        o_ref[...] = z

    return pl.pallas_call(
        body,
        out_shape=jax.ShapeDtypeStruct((_NG, 4), jnp.float32),
    )(spool, bcnt, mW0, mb0, mW1, mb1, mW2, mb2, mW3, mb3)


def kernel(x1, edge_index1, batch1, x2, edge_index2, batch2,
           gW1_0, gb1_0, gW1_1, gb1_1, gW2_0, gb2_0, gW2_1, gb2_1,
           mW0, mb0, mW1, mb1, mW2, mb2, mW3, mb3):
    pad = jnp.arange(_EP - _E, dtype=jnp.int32) % _NPAD
    src = jnp.concatenate([edge_index1[0], pad,
                           edge_index2[0] + _N, _N + pad])
    dstp = jnp.concatenate([edge_index1[1], _N + pad,
                            edge_index2[1], _N + pad])
    bat = jnp.concatenate([batch1, batch2])
    batf = bat.astype(jnp.float32).reshape(_NBLK, 1, _BLK)
    x_flat = jnp.concatenate([x1, x2])

    ones_rows = jnp.ones((_CHB, _D), jnp.float32)
    zerosD = jnp.zeros((_CHB, _D), jnp.float32)

    w0 = jnp.stack([gW1_0, gW2_0])
    w1 = jnp.stack([gW1_1, gW2_1])
    b0 = jnp.stack([gb1_0, gb2_0])[:, None, :]
    b1 = jnp.stack([gb1_1, gb2_1])[:, None, :]

    cnt, bcnt = _sc_counts(dstp, bat, ones_rows, zerosD)
    h = _tc_matmul(x_flat, w0)
    xs, dis = _tc_scale(cnt, h)
    acc1 = _sc_conv(xs, src, dstp, zerosD)
    xs2 = _tc_layer(acc1, xs, dis, b0, w1)
    acc2 = _sc_conv(xs2, src, dstp, zerosD)
    spool = _tc_pool(acc2, xs2, dis, b1, batf)
    return _tc_head(spool, bcnt, mW0, mb0, mW1, mb1, mW2, mb2, mW3, mb3)


# speedup vs baseline: 1.1956x; 1.1956x over previous
import functools

import jax
import jax.numpy as jnp
from jax import lax
from jax.experimental import pallas as pl
from jax.experimental.pallas import tpu as pltpu
from jax.experimental.pallas import tpu_sc as plsc

_N = 10000
_E = 320000
_D = 128
_NG = 64
_NC = 2
_NS = 16
_CH = 80
_ET = _E // _NS
_NCHUNK = _ET // _CH
_RCH = 80
_NROWCH = _N // _RCH
_ROW_ITERS = -(-_NROWCH // _NS)

_CHB = 128
_CHC = 64
_EPT = 20480
_EP = _NS * _EPT
_CPT = _EPT // _CHB
_CPC = _EPT // _CHC
_NPAD = 512

_HIGHEST = lax.Precision.HIGHEST


def _dot_bf16(a, b):
    return jnp.dot(a.astype(jnp.bfloat16), b.astype(jnp.bfloat16),
                   preferred_element_type=jnp.float32)

_sc_mesh = plsc.VectorSubcoreMesh(core_axis_name="c", subcore_axis_name="s")


def _sc_counts(dst, bat, ones_rows, zeros_rows):
    @functools.partial(
        pl.kernel,
        out_type=[
            jax.ShapeDtypeStruct((_NC * _N, _D), jnp.float32),
            jax.ShapeDtypeStruct((_NC * _NG, _D), jnp.float32),
        ],
        mesh=_sc_mesh,
        scratch_types=[
            pltpu.VMEM((_CHB,), jnp.int32),
            pltpu.VMEM((_CHB,), jnp.int32),
            pltpu.VMEM((_CHB,), jnp.int32),
            pltpu.VMEM((_CHB,), jnp.int32),
            pltpu.VMEM((_CH,), jnp.int32),
            pltpu.VMEM((_CHB, _D), jnp.float32),
            pltpu.VMEM_SHARED((_N + _NPAD, _D), jnp.float32),
            pltpu.VMEM_SHARED((_NG, _D), jnp.float32),
            pltpu.SemaphoreType.DMA,
            pltpu.SemaphoreType.DMA,
            pltpu.SemaphoreType.DMA,
            pltpu.SemaphoreType.DMA,
            pltpu.SemaphoreType.DMA,
            pltpu.SemaphoreType.DMA,
            pltpu.SemaphoreType.DMA,
            pltpu.SemaphoreType.DMA,
        ],
    )
    def k(dst_hbm, bat_hbm, ones_hbm, zer_hbm, cnt_hbm, bcnt_hbm,
          di0, di1, di2, di3, bidx, ones_v, cnt_s, bcnt_s,
          smi0, smi1, smi2, smi3, ss0, ss1, ss2, ss3):
        didx = (di0, di1, di2, di3)
        semi = (smi0, smi1, smi2, smi3)
        sems = (ss0, ss1, ss2, ss3)
        c = lax.axis_index("c")
        t = lax.axis_index("s")
        pltpu.sync_copy(zer_hbm, ones_v)

        @pl.loop(0, _ROW_ITERS)
        def _(i):
            ch = i * _NS + t

            @pl.when(ch < _NROWCH)
            def _():
                pltpu.sync_copy(ones_v.at[pl.ds(0, _RCH)],
                                cnt_s.at[pl.ds(ch * _RCH, _RCH)])

        @pl.when(t == 0)
        def _():
            pltpu.sync_copy(ones_v.at[pl.ds(0, _NG)], bcnt_s)

        @pl.when(t < _NPAD // _CHB)
        def _():
            pltpu.sync_copy(ones_v, cnt_s.at[pl.ds(_N + t * _CHB, _CHB)])

        pltpu.sync_copy(ones_hbm, ones_v)
        plsc.subcore_barrier()

        base_e = c * _EP + t * _EPT

        def start_idx(ch_i, j):
            pltpu.async_copy(dst_hbm.at[pl.ds(base_e + ch_i * _CHB, _CHB)],
                             didx[j], semi[j])

        def wait_idx(j):
            pltpu.make_async_copy(dst_hbm.at[pl.ds(0, _CHB)], didx[j],
                                  semi[j]).wait()

        def start_scatter(b):
            pltpu.async_copy(ones_v, cnt_s.at[didx[b]], sems[b], add=True)

        def wait_scatter(b):
            pltpu.make_async_copy(ones_v, cnt_s.at[didx[b]], sems[b]).wait()

        for b in range(4):
            start_idx(b, b)

        _G = _CPT // 4

        @pl.loop(0, _G)
        def _(g):
            for b in range(4):
                wait_idx(b)
                start_scatter(b)
            for b in range(4):
                wait_scatter(b)

                @pl.when(g < _G - 1)
                def _(b=b):
                    start_idx((g + 1) * 4 + b, b)

        @pl.loop(0, _ROW_ITERS)
        def _(i):
            ch = i * _NS + t

            @pl.when(ch < _NROWCH)
            def _():
                pltpu.sync_copy(bat_hbm.at[pl.ds(c * _N + ch * _CH, _CH)], bidx)
                pltpu.sync_copy(ones_v.at[pl.ds(0, _CH)], bcnt_s.at[bidx],
                                add=True)

        plsc.subcore_barrier()

        @pl.loop(0, _ROW_ITERS)
        def _(i):
            ch = i * _NS + t

            @pl.when(ch < _NROWCH)
            def _():
                pltpu.sync_copy(cnt_s.at[pl.ds(ch * _RCH, _RCH)],
                                cnt_hbm.at[pl.ds(c * _N + ch * _RCH, _RCH)])

        @pl.when(t == 0)
        def _():
            pltpu.sync_copy(bcnt_s, bcnt_hbm.at[pl.ds(c * _NG, _NG)])

    return k(dst, bat, ones_rows, zeros_rows)


def _sc_conv(xs_flat, src_pad, dst_pad, zeros_rows):
    @functools.partial(
        pl.kernel,
        out_type=jax.ShapeDtypeStruct((_NC * _N, _D), jnp.float32),
        mesh=_sc_mesh,
        scratch_types=(
            [pltpu.VMEM((_CHC,), jnp.int32)] * 16
            + [pltpu.VMEM((_CHC, _D), jnp.float32)] * 4
            + [pltpu.VMEM_SHARED((_N + _NPAD, _D), jnp.float32)]
            + [pltpu.SemaphoreType.DMA] * 16
        ),
    )
    def k(xs_hbm, src_hbm, dst_hbm, zer_hbm, acc_hbm, *scr):
        sidx = scr[0:8]
        didx = scr[8:16]
        rows = scr[16:20]
        acc_s = scr[20]
        semi = scr[21:29]
        semg = scr[29:33]
        sems = scr[33:37]
        c = lax.axis_index("c")
        t = lax.axis_index("s")
        @pl.loop(0, _ROW_ITERS)
        def _(i):
            ch = i * _NS + t

            @pl.when(ch < _NROWCH)
            def _():
                pltpu.sync_copy(zer_hbm.at[pl.ds(0, _RCH)],
                                acc_s.at[pl.ds(ch * _RCH, _RCH)])

        @pl.when(t < _NPAD // _CHB)
        def _():
            pltpu.sync_copy(zer_hbm, acc_s.at[pl.ds(_N + t * _CHB, _CHB)])

        plsc.subcore_barrier()

        base_e = c * _EP + t * _EPT
        TOT = _CPC

        def start_idx(ch_i, j):
            off = base_e + ch_i * _CHC
            pltpu.async_copy(src_hbm.at[pl.ds(off, _CHC)], sidx[j], semi[j])
            pltpu.async_copy(dst_hbm.at[pl.ds(off, _CHC)], didx[j], semi[j])

        def wait_idx(j):
            pltpu.make_async_copy(src_hbm.at[pl.ds(0, _CHC)], sidx[j],
                                  semi[j]).wait()
            pltpu.make_async_copy(dst_hbm.at[pl.ds(0, _CHC)], didx[j],
                                  semi[j]).wait()

        def start_gather(j, b):
            pltpu.async_copy(xs_hbm.at[sidx[j]], rows[b], semg[b])

        def wait_gather(j, b):
            pltpu.make_async_copy(xs_hbm.at[sidx[j]], rows[b], semg[b]).wait()

        def start_scatter(j, b):
            pltpu.async_copy(rows[b], acc_s.at[didx[j]], sems[b], add=True)

        def wait_scatter(j, b):
            pltpu.make_async_copy(rows[b], acc_s.at[didx[j]], sems[b]).wait()

        def step(ci):
            j, b = ci % 8, ci % 4
            if ci >= 4:
                wait_scatter((ci - 4) % 8, b)
            if ci + 4 < TOT:
                start_idx(ci + 4, (ci + 4) % 8)
            wait_idx(j)
            start_gather(j, b)
            if ci >= 2:
                wait_gather((ci - 2) % 8, (ci - 2) % 4)
                start_scatter((ci - 2) % 8, (ci - 2) % 4)

        for j in range(4):
            start_idx(j, j)
        for ci in range(8):
            step(ci)

        @pl.loop(1, TOT // 8)
        def _(g):
            for b8 in range(8):
                j, b = b8 % 8, b8 % 4
                wait_scatter((b8 + 4) % 8, b)

                @pl.when(g * 8 + b8 + 4 < TOT)
                def _(g=g, b8=b8):
                    start_idx(g * 8 + b8 + 4, (b8 + 4) % 8)

                wait_idx(j)
                start_gather(j, b)
                wait_gather((b8 + 6) % 8, (b8 + 2) % 4)
                start_scatter((b8 + 6) % 8, (b8 + 2) % 4)

        for cc in range(TOT - 2, TOT):
            wait_gather(cc % 8, cc % 4)
            start_scatter(cc % 8, cc % 4)
        for cc in range(TOT - 4, TOT):
            wait_scatter(cc % 8, cc % 4)

        plsc.subcore_barrier()

        @pl.loop(0, _ROW_ITERS)
        def _(i):
            ch = i * _NS + t

            @pl.when(ch < _NROWCH)
            def _():
                pltpu.sync_copy(acc_s.at[pl.ds(ch * _RCH, _RCH)],
                                acc_hbm.at[pl.ds(c * _N + ch * _RCH, _RCH)])

    return k(xs_flat, src_pad, dst_pad, zeros_rows)


_BLK = 2000
_NBLK = _NC * _N // _BLK
_TBLK = _N // _BLK

_row_spec = lambda w: pl.BlockSpec((_BLK, w), lambda i: (i, 0))
_pair_spec2 = pl.BlockSpec((1, 1, _D), lambda i: (i // _TBLK, 0, 0))
_pair_spec3 = pl.BlockSpec((1, _D, _D), lambda i: (i // _TBLK, 0, 0))


def _tc_matmul(x_flat, w_pair):
    def body(x_ref, w_ref, o_ref):
        o_ref[...] = _dot_bf16(x_ref[...], w_ref[0].T)

    return pl.pallas_call(
        body,
        grid=(_NBLK,),
        in_specs=[_row_spec(_D), _pair_spec3],
        out_specs=_row_spec(_D),
        out_shape=jax.ShapeDtypeStruct((_NC * _N, _D), jnp.float32),
    )(x_flat, w_pair)


def _tc_scale(cnt, h_flat):
    def body(cnt_ref, h_ref, xs_ref, dis_ref):
        dis = lax.rsqrt(cnt_ref[:, 0:1] + 1.0)
        dis_ref[...] = dis
        xs_ref[...] = h_ref[...] * dis

    return pl.pallas_call(
        body,
        grid=(_NBLK,),
        in_specs=[_row_spec(_D), _row_spec(_D)],
        out_specs=[_row_spec(_D), _row_spec(1)],
        out_shape=[
            jax.ShapeDtypeStruct((_NC * _N, _D), jnp.float32),
            jax.ShapeDtypeStruct((_NC * _N, 1), jnp.float32),
        ],
    )(cnt, h_flat)


def _tc_layer(acc_flat, xs_flat, dis, b_pair, w_pair):
    def body(acc_ref, xs_ref, dis_ref, b_ref, w_ref, o_ref):
        d = dis_ref[...]
        o = jax.nn.relu(d * (acc_ref[...] + xs_ref[...]) + b_ref[0])
        h2 = _dot_bf16(o, w_ref[0].T)
        o_ref[...] = d * h2

    return pl.pallas_call(
        body,
        grid=(_NBLK,),
        in_specs=[_row_spec(_D), _row_spec(_D), _row_spec(1),
                  _pair_spec2, _pair_spec3],
        out_specs=_row_spec(_D),
        out_shape=jax.ShapeDtypeStruct((_NC * _N, _D), jnp.float32),
    )(acc_flat, xs_flat, dis, b_pair, w_pair)


def _tc_pool(acc_flat, xs_flat, dis, b_pair, batf):
    def body(acc_ref, xs_ref, dis_ref, b_ref, bat_ref, s_ref):
        i = pl.program_id(0)
        d = dis_ref[...]
        o = jax.nn.relu(d * (acc_ref[...] + xs_ref[...]) + b_ref[0])
        seg = lax.broadcasted_iota(jnp.int32, (_NG, 1), 0).astype(jnp.float32)
        mask = (bat_ref[0] == seg).astype(jnp.float32)
        s = jnp.dot(mask, o, preferred_element_type=jnp.float32,
                    precision=_HIGHEST)

        @pl.when(i % _TBLK == 0)
        def _():
            s_ref[...] = jnp.zeros_like(s_ref)

        s_ref[0] += s

    return pl.pallas_call(
        body,
        grid=(_NBLK,),
        in_specs=[_row_spec(_D), _row_spec(_D), _row_spec(1), _pair_spec2,
                  pl.BlockSpec((1, 1, _BLK), lambda i: (i, 0, 0))],
        out_specs=pl.BlockSpec((1, _NG, _D), lambda i: (i // _TBLK, 0, 0)),
        out_shape=jax.ShapeDtypeStruct((_NC, _NG, _D), jnp.float32),
    )(acc_flat, xs_flat, dis, b_pair, batf)


def _tc_head(spool, bcnt, mW0, mb0, mW1, mb1, mW2, mb2, mW3, mb3):
    def body(s_ref, bcnt_ref, w0_ref, b0_ref, w1_ref, b1_ref, w2_ref, b2_ref,
             w3_ref, b3_ref, o_ref):
        g1 = s_ref[0] / jnp.maximum(bcnt_ref[0:_NG, 0:1], 1.0)
        g2 = s_ref[1] / jnp.maximum(bcnt_ref[_NG:2 * _NG, 0:1], 1.0)
        z = jnp.concatenate([g1, g2], axis=1)
        for w_ref, bias_ref, act in ((w0_ref, b0_ref, True), (w1_ref, b1_ref, True),
                                     (w2_ref, b2_ref, True), (w3_ref, b3_ref, False)):
            z = _dot_bf16(z, w_ref[...].T) + bias_ref[...][None, :]
            if act:
                z = jax.nn.relu(z)
        o_ref[...] = z

    return pl.pallas_call(
        body,
        out_shape=jax.ShapeDtypeStruct((_NG, 4), jnp.float32),
    )(spool, bcnt, mW0, mb0, mW1, mb1, mW2, mb2, mW3, mb3)


def kernel(x1, edge_index1, batch1, x2, edge_index2, batch2,
           gW1_0, gb1_0, gW1_1, gb1_1, gW2_0, gb2_0, gW2_1, gb2_1,
           mW0, mb0, mW1, mb1, mW2, mb2, mW3, mb3):
    npt = _EPT - _ET
    prng = jnp.arange(npt, dtype=jnp.int32)[None, :]
    tid = jnp.arange(_NS, dtype=jnp.int32)[:, None]
    pad_src = (tid * npt + prng) % _N
    pad_dst = _N + (tid * 37 + prng) % _NPAD

    def _pad_tower(e, off):
        return jnp.concatenate(
            [e.reshape(_NS, _ET) + off, pad_src + off], axis=1).reshape(-1)

    src = jnp.concatenate([_pad_tower(edge_index1[0], 0),
                           _pad_tower(edge_index2[0], _N)])

    def _pad_dst_tower(e):
        return jnp.concatenate(
            [e.reshape(_NS, _ET), pad_dst], axis=1).reshape(-1)

    dstp = jnp.concatenate([_pad_dst_tower(edge_index1[1]),
                            _pad_dst_tower(edge_index2[1])])
    bat = jnp.concatenate([batch1, batch2])
    batf = bat.astype(jnp.float32).reshape(_NBLK, 1, _BLK)
    x_flat = jnp.concatenate([x1, x2])

    ones_rows = jnp.ones((_CHB, _D), jnp.float32)
    zerosD = jnp.zeros((_CHB, _D), jnp.float32)

    w0 = jnp.stack([gW1_0, gW2_0])
    w1 = jnp.stack([gW1_1, gW2_1])
    b0 = jnp.stack([gb1_0, gb2_0])[:, None, :]
    b1 = jnp.stack([gb1_1, gb2_1])[:, None, :]

    cnt, bcnt = _sc_counts(dstp, bat, ones_rows, zerosD)
    h = _tc_matmul(x_flat, w0)
    xs, dis = _tc_scale(cnt, h)
    acc1 = _sc_conv(xs, src, dstp, zerosD)
    xs2 = _tc_layer(acc1, xs, dis, b0, w1)
    acc2 = _sc_conv(xs2, src, dstp, zerosD)
    spool = _tc_pool(acc2, xs2, dis, b1, batf)
    return _tc_head(spool, bcnt, mW0, mb0, mW1, mb1, mW2, mb2, mW3, mb3)


# speedup vs baseline: 1.2075x; 1.0099x over previous
import functools

import jax
import jax.numpy as jnp
from jax import lax
from jax.experimental import pallas as pl
from jax.experimental.pallas import tpu as pltpu
from jax.experimental.pallas import tpu_sc as plsc

_N = 10000
_E = 320000
_D = 128
_NG = 64
_NC = 2
_NS = 16
_CH = 80
_ET = _E // _NS
_NCHUNK = _ET // _CH
_RCH = 80
_NROWCH = _N // _RCH
_ROW_ITERS = -(-_NROWCH // _NS)

_CHB = 128
_CHC = 80
_EPT = 20480
_EP = _NS * _EPT
_CPT = _EPT // _CHB
_CPC = _EPT // _CHC
_NPAD = 256

_HIGHEST = lax.Precision.HIGHEST


def _dot_bf16(a, b):
    return jnp.dot(a.astype(jnp.bfloat16), b.astype(jnp.bfloat16),
                   preferred_element_type=jnp.float32)

_sc_mesh = plsc.VectorSubcoreMesh(core_axis_name="c", subcore_axis_name="s")


def _sc_counts(dst, bat, ones_rows, zeros_rows):
    @functools.partial(
        pl.kernel,
        out_type=[
            jax.ShapeDtypeStruct((_NC * _N, _D), jnp.float32),
            jax.ShapeDtypeStruct((_NC * _NG, _D), jnp.float32),
        ],
        mesh=_sc_mesh,
        scratch_types=[
            pltpu.VMEM((_CHB,), jnp.int32),
            pltpu.VMEM((_CHB,), jnp.int32),
            pltpu.VMEM((_CHB,), jnp.int32),
            pltpu.VMEM((_CHB,), jnp.int32),
            pltpu.VMEM((_CH,), jnp.int32),
            pltpu.VMEM((_CHB, _D), jnp.float32),
            pltpu.VMEM_SHARED((_N + _NPAD, _D), jnp.float32),
            pltpu.VMEM_SHARED((_NG, _D), jnp.float32),
            pltpu.SemaphoreType.DMA,
            pltpu.SemaphoreType.DMA,
            pltpu.SemaphoreType.DMA,
            pltpu.SemaphoreType.DMA,
            pltpu.SemaphoreType.DMA,
            pltpu.SemaphoreType.DMA,
            pltpu.SemaphoreType.DMA,
            pltpu.SemaphoreType.DMA,
        ],
    )
    def k(dst_hbm, bat_hbm, ones_hbm, zer_hbm, cnt_hbm, bcnt_hbm,
          di0, di1, di2, di3, bidx, ones_v, cnt_s, bcnt_s,
          smi0, smi1, smi2, smi3, ss0, ss1, ss2, ss3):
        didx = (di0, di1, di2, di3)
        semi = (smi0, smi1, smi2, smi3)
        sems = (ss0, ss1, ss2, ss3)
        c = lax.axis_index("c")
        t = lax.axis_index("s")
        pltpu.sync_copy(zer_hbm, ones_v)

        @pl.loop(0, _ROW_ITERS)
        def _(i):
            ch = i * _NS + t

            @pl.when(ch < _NROWCH)
            def _():
                pltpu.sync_copy(ones_v.at[pl.ds(0, _RCH)],
                                cnt_s.at[pl.ds(ch * _RCH, _RCH)])

        @pl.when(t == 0)
        def _():
            pltpu.sync_copy(ones_v.at[pl.ds(0, _NG)], bcnt_s)

        @pl.when(t < _NPAD // _CHB)
        def _():
            pltpu.sync_copy(ones_v, cnt_s.at[pl.ds(_N + t * _CHB, _CHB)])

        pltpu.sync_copy(ones_hbm, ones_v)
        plsc.subcore_barrier()

        base_e = c * _EP + t * _EPT

        def start_idx(ch_i, j):
            pltpu.async_copy(dst_hbm.at[pl.ds(base_e + ch_i * _CHB, _CHB)],
                             didx[j], semi[j])

        def wait_idx(j):
            pltpu.make_async_copy(dst_hbm.at[pl.ds(0, _CHB)], didx[j],
                                  semi[j]).wait()

        def start_scatter(b):
            pltpu.async_copy(ones_v, cnt_s.at[didx[b]], sems[b], add=True)

        def wait_scatter(b):
            pltpu.make_async_copy(ones_v, cnt_s.at[didx[b]], sems[b]).wait()

        for b in range(4):
            start_idx(b, b)

        _G = _CPT // 4

        @pl.loop(0, _G)
        def _(g):
            for b in range(4):
                wait_idx(b)
                start_scatter(b)
            for b in range(4):
                wait_scatter(b)

                @pl.when(g < _G - 1)
                def _(b=b):
                    start_idx((g + 1) * 4 + b, b)

        @pl.loop(0, _ROW_ITERS)
        def _(i):
            ch = i * _NS + t

            @pl.when(ch < _NROWCH)
            def _():
                pltpu.sync_copy(bat_hbm.at[pl.ds(c * _N + ch * _CH, _CH)], bidx)
                pltpu.sync_copy(ones_v.at[pl.ds(0, _CH)], bcnt_s.at[bidx],
                                add=True)

        plsc.subcore_barrier()

        @pl.loop(0, _ROW_ITERS)
        def _(i):
            ch = i * _NS + t

            @pl.when(ch < _NROWCH)
            def _():
                pltpu.sync_copy(cnt_s.at[pl.ds(ch * _RCH, _RCH)],
                                cnt_hbm.at[pl.ds(c * _N + ch * _RCH, _RCH)])

        @pl.when(t == 0)
        def _():
            pltpu.sync_copy(bcnt_s, bcnt_hbm.at[pl.ds(c * _NG, _NG)])

    return k(dst, bat, ones_rows, zeros_rows)


def _sc_conv(xs_flat, src_pad, dst_pad, zeros_rows):
    @functools.partial(
        pl.kernel,
        out_type=jax.ShapeDtypeStruct((_NC * _N, _D), jnp.float32),
        mesh=_sc_mesh,
        scratch_types=(
            [pltpu.VMEM((_CHC,), jnp.int32)] * 16
            + [pltpu.VMEM((_CHC, _D), jnp.float32)] * 4
            + [pltpu.VMEM_SHARED((_N + _NPAD, _D), jnp.float32)]
            + [pltpu.SemaphoreType.DMA] * 16
        ),
    )
    def k(xs_hbm, src_hbm, dst_hbm, zer_hbm, acc_hbm, *scr):
        sidx = scr[0:8]
        didx = scr[8:16]
        rows = scr[16:20]
        acc_s = scr[20]
        semi = scr[21:29]
        semg = scr[29:33]
        sems = scr[33:37]
        c = lax.axis_index("c")
        t = lax.axis_index("s")
        @pl.loop(0, _ROW_ITERS)
        def _(i):
            ch = i * _NS + t

            @pl.when(ch < _NROWCH)
            def _():
                pltpu.sync_copy(zer_hbm.at[pl.ds(0, _RCH)],
                                acc_s.at[pl.ds(ch * _RCH, _RCH)])

        @pl.when(t < _NPAD // _CHB)
        def _():
            pltpu.sync_copy(zer_hbm, acc_s.at[pl.ds(_N + t * _CHB, _CHB)])

        plsc.subcore_barrier()

        base_e = c * _EP + t * _EPT
        TOT = _CPC

        def start_idx(ch_i, j):
            off = base_e + ch_i * _CHC
            pltpu.async_copy(src_hbm.at[pl.ds(off, _CHC)], sidx[j], semi[j])
            pltpu.async_copy(dst_hbm.at[pl.ds(off, _CHC)], didx[j], semi[j])

        def wait_idx(j):
            pltpu.make_async_copy(src_hbm.at[pl.ds(0, _CHC)], sidx[j],
                                  semi[j]).wait()
            pltpu.make_async_copy(dst_hbm.at[pl.ds(0, _CHC)], didx[j],
                                  semi[j]).wait()

        def start_gather(j, b):
            pltpu.async_copy(xs_hbm.at[sidx[j]], rows[b], semg[b])

        def wait_gather(j, b):
            pltpu.make_async_copy(xs_hbm.at[sidx[j]], rows[b], semg[b]).wait()

        def start_scatter(j, b):
            pltpu.async_copy(rows[b], acc_s.at[didx[j]], sems[b], add=True)

        def wait_scatter(j, b):
            pltpu.make_async_copy(rows[b], acc_s.at[didx[j]], sems[b]).wait()

        def step(ci):
            j, b = ci % 8, ci % 4
            if ci >= 4:
                wait_scatter((ci - 4) % 8, b)
            if ci + 4 < TOT:
                start_idx(ci + 4, (ci + 4) % 8)
            wait_idx(j)
            start_gather(j, b)
            if ci >= 2:
                wait_gather((ci - 2) % 8, (ci - 2) % 4)
                start_scatter((ci - 2) % 8, (ci - 2) % 4)

        for j in range(4):
            start_idx(j, j)
        for ci in range(8):
            step(ci)

        @pl.loop(1, TOT // 8)
        def _(g):
            for b8 in range(8):
                j, b = b8 % 8, b8 % 4
                wait_scatter((b8 + 4) % 8, b)

                @pl.when(g * 8 + b8 + 4 < TOT)
                def _(g=g, b8=b8):
                    start_idx(g * 8 + b8 + 4, (b8 + 4) % 8)

                wait_idx(j)
                start_gather(j, b)
                wait_gather((b8 + 6) % 8, (b8 + 2) % 4)
                start_scatter((b8 + 6) % 8, (b8 + 2) % 4)

        for cc in range(TOT - 2, TOT):
            wait_gather(cc % 8, cc % 4)
            start_scatter(cc % 8, cc % 4)
        for cc in range(TOT - 4, TOT):
            wait_scatter(cc % 8, cc % 4)

        plsc.subcore_barrier()

        @pl.loop(0, _ROW_ITERS)
        def _(i):
            ch = i * _NS + t

            @pl.when(ch < _NROWCH)
            def _():
                pltpu.sync_copy(acc_s.at[pl.ds(ch * _RCH, _RCH)],
                                acc_hbm.at[pl.ds(c * _N + ch * _RCH, _RCH)])

    return k(xs_flat, src_pad, dst_pad, zeros_rows)


_BLK = 2000
_NBLK = _NC * _N // _BLK
_TBLK = _N // _BLK

_row_spec = lambda w: pl.BlockSpec((_BLK, w), lambda i: (i, 0))
_pair_spec2 = pl.BlockSpec((1, 1, _D), lambda i: (i // _TBLK, 0, 0))
_pair_spec3 = pl.BlockSpec((1, _D, _D), lambda i: (i // _TBLK, 0, 0))


def _tc_matmul(x_flat, w_pair):
    def body(x_ref, w_ref, o_ref):
        o_ref[...] = _dot_bf16(x_ref[...], w_ref[0].T)

    return pl.pallas_call(
        body,
        grid=(_NBLK,),
        in_specs=[_row_spec(_D), _pair_spec3],
        out_specs=_row_spec(_D),
        out_shape=jax.ShapeDtypeStruct((_NC * _N, _D), jnp.float32),
    )(x_flat, w_pair)


def _tc_scale(cnt, h_flat):
    def body(cnt_ref, h_ref, xs_ref, dis_ref):
        dis = lax.rsqrt(cnt_ref[:, 0:1] + 1.0)
        dis_ref[...] = dis
        xs_ref[...] = h_ref[...] * dis

    return pl.pallas_call(
        body,
        grid=(_NBLK,),
        in_specs=[_row_spec(_D), _row_spec(_D)],
        out_specs=[_row_spec(_D), _row_spec(1)],
        out_shape=[
            jax.ShapeDtypeStruct((_NC * _N, _D), jnp.float32),
            jax.ShapeDtypeStruct((_NC * _N, 1), jnp.float32),
        ],
    )(cnt, h_flat)


def _tc_layer(acc_flat, xs_flat, dis, b_pair, w_pair):
    def body(acc_ref, xs_ref, dis_ref, b_ref, w_ref, o_ref):
        d = dis_ref[...]
        o = jax.nn.relu(d * (acc_ref[...] + xs_ref[...]) + b_ref[0])
        h2 = _dot_bf16(o, w_ref[0].T)
        o_ref[...] = d * h2

    return pl.pallas_call(
        body,
        grid=(_NBLK,),
        in_specs=[_row_spec(_D), _row_spec(_D), _row_spec(1),
                  _pair_spec2, _pair_spec3],
        out_specs=_row_spec(_D),
        out_shape=jax.ShapeDtypeStruct((_NC * _N, _D), jnp.float32),
    )(acc_flat, xs_flat, dis, b_pair, w_pair)


def _tc_pool(acc_flat, xs_flat, dis, b_pair, batf):
    def body(acc_ref, xs_ref, dis_ref, b_ref, bat_ref, s_ref):
        i = pl.program_id(0)
        d = dis_ref[...]
        o = jax.nn.relu(d * (acc_ref[...] + xs_ref[...]) + b_ref[0])
        seg = lax.broadcasted_iota(jnp.int32, (_NG, 1), 0).astype(jnp.float32)
        mask = (bat_ref[0] == seg).astype(jnp.float32)
        s = jnp.dot(mask, o, preferred_element_type=jnp.float32,
                    precision=_HIGHEST)

        @pl.when(i % _TBLK == 0)
        def _():
            s_ref[...] = jnp.zeros_like(s_ref)

        s_ref[0] += s

    return pl.pallas_call(
        body,
        grid=(_NBLK,),
        in_specs=[_row_spec(_D), _row_spec(_D), _row_spec(1), _pair_spec2,
                  pl.BlockSpec((1, 1, _BLK), lambda i: (i, 0, 0))],
        out_specs=pl.BlockSpec((1, _NG, _D), lambda i: (i // _TBLK, 0, 0)),
        out_shape=jax.ShapeDtypeStruct((_NC, _NG, _D), jnp.float32),
    )(acc_flat, xs_flat, dis, b_pair, batf)


def _tc_head(spool, bcnt, mW0, mb0, mW1, mb1, mW2, mb2, mW3, mb3):
    def body(s_ref, bcnt_ref, w0_ref, b0_ref, w1_ref, b1_ref, w2_ref, b2_ref,
             w3_ref, b3_ref, o_ref):
        g1 = s_ref[0] / jnp.maximum(bcnt_ref[0:_NG, 0:1], 1.0)
        g2 = s_ref[1] / jnp.maximum(bcnt_ref[_NG:2 * _NG, 0:1], 1.0)
        z = jnp.concatenate([g1, g2], axis=1)
        for w_ref, bias_ref, act in ((w0_ref, b0_ref, True), (w1_ref, b1_ref, True),
                                     (w2_ref, b2_ref, True), (w3_ref, b3_ref, False)):
            z = _dot_bf16(z, w_ref[...].T) + bias_ref[...][None, :]
            if act:
                z = jax.nn.relu(z)
        o_ref[...] = z

    return pl.pallas_call(
        body,
        out_shape=jax.ShapeDtypeStruct((_NG, 4), jnp.float32),
    )(spool, bcnt, mW0, mb0, mW1, mb1, mW2, mb2, mW3, mb3)


def kernel(x1, edge_index1, batch1, x2, edge_index2, batch2,
           gW1_0, gb1_0, gW1_1, gb1_1, gW2_0, gb2_0, gW2_1, gb2_1,
           mW0, mb0, mW1, mb1, mW2, mb2, mW3, mb3):
    npt = _EPT - _ET
    prng = jnp.arange(npt, dtype=jnp.int32)[None, :]
    tid = jnp.arange(_NS, dtype=jnp.int32)[:, None]
    pad_src = (tid * npt + prng) % _N
    pad_dst = _N + (tid * 37 + prng) % _NPAD

    def _pad_tower(e, off):
        return jnp.concatenate(
            [e.reshape(_NS, _ET) + off, pad_src + off], axis=1).reshape(-1)

    src = jnp.concatenate([_pad_tower(edge_index1[0], 0),
                           _pad_tower(edge_index2[0], _N)])

    def _pad_dst_tower(e):
        return jnp.concatenate(
            [e.reshape(_NS, _ET), pad_dst], axis=1).reshape(-1)

    dstp = jnp.concatenate([_pad_dst_tower(edge_index1[1]),
                            _pad_dst_tower(edge_index2[1])])
    bat = jnp.concatenate([batch1, batch2])
    batf = bat.astype(jnp.float32).reshape(_NBLK, 1, _BLK)
    x_flat = jnp.concatenate([x1, x2])

    ones_rows = jnp.ones((_CHB, _D), jnp.float32)
    zerosD = jnp.zeros((_CHB, _D), jnp.float32)

    w0 = jnp.stack([gW1_0, gW2_0])
    w1 = jnp.stack([gW1_1, gW2_1])
    b0 = jnp.stack([gb1_0, gb2_0])[:, None, :]
    b1 = jnp.stack([gb1_1, gb2_1])[:, None, :]

    cnt, bcnt = _sc_counts(dstp, bat, ones_rows, zerosD)
    h = _tc_matmul(x_flat, w0)
    xs, dis = _tc_scale(cnt, h)
    acc1 = _sc_conv(xs, src, dstp, zerosD)
    xs2 = _tc_layer(acc1, xs, dis, b0, w1)
    acc2 = _sc_conv(xs2, src, dstp, zerosD)
    spool = _tc_pool(acc2, xs2, dis, b1, batf)
    return _tc_head(spool, bcnt, mW0, mb0, mW1, mb1, mW2, mb2, mW3, mb3)
